# Initial kernel scaffold; baseline (speedup 1.0000x reference)
import jax
import jax.numpy as jnp
from jax.experimental import pallas as pl


def kernel(x, pos_edge_index, neg_edge_index, Wp, bp, w1_pl, w1_pr, b1_pr, w1_nl, w1_nr, b1_nr, w2_pl, w2_pr, b2_pr, w2_nl, w2_nr, b2_nr):
    raise NotImplementedError("write your pallas kernel here")


# speedup vs baseline: 3.9167x; 3.9167x over previous
import functools

import jax
import jax.numpy as jnp
from jax import lax
from jax.experimental import pallas as pl
from jax.experimental.pallas import tpu as pltpu
from jax.experimental.pallas import tpu_sc as plsc

N = 50000
D = 128
F2 = 64
C = 32
NCH = 4
E = 400000
B = 128
KB = 196
NT = 16
EP = NT * KB * B
PAD = EP - E
ACC_ROWS = 50048
ZROWS = 1564
RPT = ACC_ROWS // NT
NP = ACC_ROWS
BN = 1000


GB = 14


def _make_agg():
  mesh = plsc.VectorSubcoreMesh(core_axis_name="c", subcore_axis_name="s")
  out_type = [jax.ShapeDtypeStruct((2, NP, C), jnp.float32)
              for _ in range(NCH)]
  scratch = [
      pltpu.VMEM((GB, B), jnp.int32),
      pltpu.VMEM((GB, B), jnp.int32),
      pltpu.VMEM((B, C), jnp.float32),
      pltpu.VMEM((B, C), jnp.float32),
      pltpu.VMEM_SHARED((ACC_ROWS, C), jnp.float32),
      pltpu.SemaphoreType.DMA,
      pltpu.SemaphoreType.DMA,
      pltpu.SemaphoreType.DMA,
      pltpu.SemaphoreType.DMA,
  ]

  @functools.partial(pl.kernel, out_type=out_type, mesh=mesh,
                     scratch_types=scratch,
                     compiler_params=pltpu.CompilerParams(
                         use_tc_tiling_on_sc=False))
  def agg(t0, t1, t2, t3, src_all, dst_all, zeros_hbm,
          o0, o1, o2, o3, isrc, idst, rb0, rb1,
          acc, gs0, gs1, ss0, ss1):
    outs = (o0, o1, o2, o3)
    tables = (t0, t1, t2, t3)
    rbufs = (rb0, rb1)
    gsems = (gs0, gs1)
    ssems = (ss0, ss1)
    core = lax.axis_index("c")
    s = lax.axis_index("s")
    row0 = s * RPT
    for ch in range(NCH):
      pltpu.sync_copy(zeros_hbm, acc.at[pl.ds(row0, RPT), :])
      plsc.subcore_barrier()
      t = tables[ch]

      def group(g, carry, t=t):
        pltpu.sync_copy(src_all.at[core, s, pl.ds(g * GB, GB)], isrc)
        pltpu.sync_copy(dst_all.at[core, s, pl.ds(g * GB, GB)], idst)
        dg = [None] * GB
        dsc = [None] * GB
        dg[0] = pltpu.async_copy(t.at[isrc.at[0]], rb0, gs0)
        for j in range(GB):
          a = j % 2
          dg[j].wait()
          if j >= 1:
            dsc[j - 1].wait()
          if j + 1 < GB:
            nb = (j + 1) % 2
            dg[j + 1] = pltpu.async_copy(t.at[isrc.at[j + 1]],
                                         rbufs[nb], gsems[nb])
          dsc[j] = pltpu.async_copy(rbufs[a], acc.at[idst.at[j]],
                                    ssems[a], add=True)
        dsc[GB - 1].wait()
        return carry

      lax.fori_loop(0, KB // GB, group, 0)
      plsc.subcore_barrier()
      pltpu.sync_copy(acc.at[pl.ds(row0, RPT), :],
                      outs[ch].at[core, pl.ds(row0, RPT), :])

  return agg


def _make_counts():
  mesh = plsc.VectorSubcoreMesh(core_axis_name="c", subcore_axis_name="s")
  scratch = [
      pltpu.VMEM((GB, B), jnp.int32),
      pltpu.VMEM((B, 8), jnp.float32),
      pltpu.VMEM_SHARED((ACC_ROWS, 8), jnp.float32),
  ]

  @functools.partial(pl.kernel,
                     out_type=jax.ShapeDtypeStruct((2, NP, 8), jnp.float32),
                     mesh=mesh, scratch_types=scratch,
                     compiler_params=pltpu.CompilerParams(
                         use_tc_tiling_on_sc=False))
  def cnt(dst_all, zeros8_hbm, ones8_hbm, cnt_out, idst, ones_v, cacc):
    core = lax.axis_index("c")
    s = lax.axis_index("s")
    row0 = s * RPT
    pltpu.sync_copy(ones8_hbm, ones_v)
    pltpu.sync_copy(zeros8_hbm, cacc.at[pl.ds(row0, RPT), :])
    plsc.subcore_barrier()

    def group(g, carry):
      pltpu.sync_copy(dst_all.at[core, s, pl.ds(g * GB, GB)], idst)
      for j in range(GB):
        pltpu.sync_copy(ones_v, cacc.at[idst.at[j]], add=True)
      return carry

    lax.fori_loop(0, KB // GB, group, 0)
    plsc.subcore_barrier()
    pltpu.sync_copy(cacc.at[pl.ds(row0, RPT), :],
                    cnt_out.at[core, pl.ds(row0, RPT), :])

  return cnt


def _proj(x, Wp, bp2):
  def body(x_ref, w_ref, b_ref, o0, o1, o2, o3):
    y = jnp.dot(x_ref[...], w_ref[...],
                preferred_element_type=jnp.float32) + b_ref[...]
    for c, o in enumerate((o0, o1, o2, o3)):
      o[...] = y[:, C * c:C * (c + 1)]
  return pl.pallas_call(
      body,
      grid=(N // BN,),
      in_specs=[pl.BlockSpec((BN, D), lambda i: (i, 0)),
                pl.BlockSpec((D, D), lambda i: (0, 0)),
                pl.BlockSpec((1, D), lambda i: (0, 0))],
      out_specs=[pl.BlockSpec((BN, C), lambda i: (i, 0))] * NCH,
      out_shape=[jax.ShapeDtypeStruct((N, C), jnp.float32)] * NCH,
  )(x, Wp, bp2)


def _mix1(hs, As, cnt8, w1_pl, w1_pr, b1_pr2, w1_nl, w1_nr, b1_nr2):
  def body(h0, h1, h2, h3, a0, a1, a2, a3, cnt_ref,
           wpl, wpr, bpr, wnl, wnr, bnr, z0, z1, z2, z3):
    cnt = cnt_ref[...]
    rp = 1.0 / jnp.maximum(cnt[0, :, 0:1], 1.0)
    rn = 1.0 / jnp.maximum(cnt[1, :, 0:1], 1.0)
    op = jnp.broadcast_to(bpr[...], (BN, F2))
    on = jnp.broadcast_to(bnr[...], (BN, F2))
    for c, (h_ref, a_ref) in enumerate(zip((h0, h1, h2, h3),
                                           (a0, a1, a2, a3))):
      sl = slice(C * c, C * (c + 1))
      a = a_ref[...]
      hc = h_ref[...]
      op = (op
            + jnp.dot(a[0] * rp, wpl[sl, :], preferred_element_type=jnp.float32)
            + jnp.dot(hc, wpr[sl, :], preferred_element_type=jnp.float32))
      on = (on
            + jnp.dot(a[1] * rn, wnl[sl, :], preferred_element_type=jnp.float32)
            + jnp.dot(hc, wnr[sl, :], preferred_element_type=jnp.float32))
    zp = jnp.tanh(op)
    zn = jnp.tanh(on)
    z0[...] = zp[:, :C]
    z1[...] = zp[:, C:]
    z2[...] = zn[:, :C]
    z3[...] = zn[:, C:]

  blk_h = pl.BlockSpec((BN, C), lambda i: (i, 0))
  blk_a = pl.BlockSpec((2, BN, C), lambda i: (0, i, 0))
  return pl.pallas_call(
      body,
      grid=(N // BN,),
      in_specs=[blk_h] * NCH + [blk_a] * NCH
      + [pl.BlockSpec((2, BN, 8), lambda i: (0, i, 0)),
         pl.BlockSpec((D, F2), lambda i: (0, 0)),
         pl.BlockSpec((D, F2), lambda i: (0, 0)),
         pl.BlockSpec((1, F2), lambda i: (0, 0)),
         pl.BlockSpec((D, F2), lambda i: (0, 0)),
         pl.BlockSpec((D, F2), lambda i: (0, 0)),
         pl.BlockSpec((1, F2), lambda i: (0, 0))],
      out_specs=[blk_h] * NCH,
      out_shape=[jax.ShapeDtypeStruct((N, C), jnp.float32)] * NCH,
  )(*hs, *As, cnt8, w1_pl, w1_pr, b1_pr2, w1_nl, w1_nr, b1_nr2)


def _mix2(zs, Bs, cnt8, w2_pl, w2_pr, b2_pr2, w2_nl, w2_nr, b2_nr2):
  def body(z0, z1, z2, z3, g0, g1, g2, g3, cnt_ref,
           wpl, wpr, bpr, wnl, wnr, bnr, out_ref):
    cnt = cnt_ref[...]
    rp = 1.0 / jnp.maximum(cnt[0, :, 0:1], 1.0)
    rn = 1.0 / jnp.maximum(cnt[1, :, 0:1], 1.0)
    g = [r[...] for r in (g0, g1, g2, g3)]
    zl = [r[...] for r in (z0, z1, z2, z3)]
    dot = functools.partial(jnp.dot, preferred_element_type=jnp.float32)
    op = (jnp.broadcast_to(bpr[...], (BN, F2))
          + dot(g[0][0] * rp, wpl[0 * C:1 * C, :])
          + dot(g[1][0] * rp, wpl[1 * C:2 * C, :])
          + dot(g[2][1] * rn, wpl[2 * C:3 * C, :])
          + dot(g[3][1] * rn, wpl[3 * C:4 * C, :])
          + dot(zl[0], wpr[0:C, :]) + dot(zl[1], wpr[C:2 * C, :]))
    on = (jnp.broadcast_to(bnr[...], (BN, F2))
          + dot(g[2][0] * rp, wnl[0 * C:1 * C, :])
          + dot(g[3][0] * rp, wnl[1 * C:2 * C, :])
          + dot(g[0][1] * rn, wnl[2 * C:3 * C, :])
          + dot(g[1][1] * rn, wnl[3 * C:4 * C, :])
          + dot(zl[2], wnr[0:C, :]) + dot(zl[3], wnr[C:2 * C, :]))
    out_ref[:, :F2] = jnp.tanh(op)
    out_ref[:, F2:] = jnp.tanh(on)

  blk_z = pl.BlockSpec((BN, C), lambda i: (i, 0))
  blk_a = pl.BlockSpec((2, BN, C), lambda i: (0, i, 0))
  return pl.pallas_call(
      body,
      grid=(N // BN,),
      in_specs=[blk_z] * NCH + [blk_a] * NCH
      + [pl.BlockSpec((2, BN, 8), lambda i: (0, i, 0)),
         pl.BlockSpec((D, F2), lambda i: (0, 0)),
         pl.BlockSpec((F2, F2), lambda i: (0, 0)),
         pl.BlockSpec((1, F2), lambda i: (0, 0)),
         pl.BlockSpec((D, F2), lambda i: (0, 0)),
         pl.BlockSpec((F2, F2), lambda i: (0, 0)),
         pl.BlockSpec((1, F2), lambda i: (0, 0))],
      out_specs=pl.BlockSpec((BN, D), lambda i: (i, 0)),
      out_shape=jax.ShapeDtypeStruct((N, D), jnp.float32),
  )(*zs, *Bs, cnt8, w2_pl, w2_pr, b2_pr2, w2_nl, w2_nr, b2_nr2)


def kernel(x, pos_edge_index, neg_edge_index, Wp, bp,
           w1_pl, w1_pr, b1_pr, w1_nl, w1_nr, b1_nr,
           w2_pl, w2_pr, b2_pr, w2_nl, w2_nr, b2_nr):
  i32 = jnp.int32
  f32 = jnp.float32
  pad_src = jnp.zeros((PAD,), i32)
  pad_dst = jnp.full((PAD,), N, i32)
  src_all = jnp.stack([
      jnp.concatenate([pos_edge_index[0].astype(i32), pad_src]),
      jnp.concatenate([neg_edge_index[0].astype(i32), pad_src]),
  ]).reshape(2, NT, KB, B)
  dst_all = jnp.stack([
      jnp.concatenate([pos_edge_index[1].astype(i32), pad_dst]),
      jnp.concatenate([neg_edge_index[1].astype(i32), pad_dst]),
  ]).reshape(2, NT, KB, B)
  zeros32 = jnp.zeros((RPT, C), f32)
  zeros8 = jnp.zeros((RPT, 8), f32)
  ones8 = jnp.ones((B, 8), f32)

  hs = _proj(x, Wp, bp.reshape(1, D))
  agg = _make_agg()
  cnt8 = _make_counts()(dst_all, zeros8, ones8)
  As = agg(*hs, src_all, dst_all, zeros32)
  zs = _mix1(hs, As, cnt8, w1_pl, w1_pr, b1_pr.reshape(1, F2),
             w1_nl, w1_nr, b1_nr.reshape(1, F2))
  Bs = agg(*zs, src_all, dst_all, zeros32)
  return _mix2(zs, Bs, cnt8, w2_pl, w2_pr, b2_pr.reshape(1, F2),
               w2_nl, w2_nr, b2_nr.reshape(1, F2))


# speedup vs baseline: 5.3982x; 1.3783x over previous
import functools

import jax
import jax.numpy as jnp
from jax import lax
from jax.experimental import pallas as pl
from jax.experimental.pallas import tpu as pltpu
from jax.experimental.pallas import tpu_sc as plsc

N = 50000
D = 128
F2 = 64
C = 32
NCH = 4
E = 400000
B = 128
KB = 196
NT = 16
EP = NT * KB * B
PAD = EP - E
ACC_ROWS = 50048
ZROWS = 1564
RPT = ACC_ROWS // NT
NP = ACC_ROWS
BN = 2000


GB = 14


def _make_agg():
  mesh = plsc.VectorSubcoreMesh(core_axis_name="c", subcore_axis_name="s")
  out_type = [jax.ShapeDtypeStruct((2, NP, C), jnp.float32)
              for _ in range(NCH)]
  scratch = [
      pltpu.VMEM((GB, B), jnp.int32),
      pltpu.VMEM((GB, B), jnp.int32),
      pltpu.VMEM((B, C), jnp.float32),
      pltpu.VMEM((B, C), jnp.float32),
      pltpu.VMEM((B, C), jnp.float32),
      pltpu.VMEM((B, C), jnp.float32),
      pltpu.VMEM_SHARED((ACC_ROWS, C), jnp.float32),
  ] + [pltpu.SemaphoreType.DMA] * 6

  @functools.partial(pl.kernel, out_type=out_type, mesh=mesh,
                     scratch_types=scratch,
                     compiler_params=pltpu.CompilerParams(
                         use_tc_tiling_on_sc=False))
  def agg(t0, t1, t2, t3, src_all, dst_all, zeros_hbm,
          o0, o1, o2, o3, isrc, idst, rb0, rb1, rb2, rb3,
          acc, gs0, gs1, gs2, gs3, ss0, ss1):
    outs = (o0, o1, o2, o3)
    tables = (t0, t1, t2, t3)
    rbufs = (rb0, rb1, rb2, rb3)
    gsems = (gs0, gs1, gs2, gs3)
    ssems = (ss0, ss1)
    core = lax.axis_index("c")
    s = lax.axis_index("s")
    row0 = s * RPT
    for ch in range(NCH):
      pltpu.sync_copy(zeros_hbm, acc.at[pl.ds(row0, RPT), :])
      plsc.subcore_barrier()
      t = tables[ch]

      def group(g, carry, t=t):
        pltpu.sync_copy(src_all.at[core, s, pl.ds(g * GB, GB)], isrc)
        pltpu.sync_copy(dst_all.at[core, s, pl.ds(g * GB, GB)], idst)
        dg = [None] * GB
        dsc = [None] * GB
        for p in range(3):
          dg[p] = pltpu.async_copy(t.at[isrc.at[p]], rbufs[p], gsems[p])
        for j in range(GB):
          a = j % 4
          dg[j].wait()
          if j >= 1:
            dsc[j - 1].wait()
          if j + 3 < GB:
            nb = (j + 3) % 4
            dg[j + 3] = pltpu.async_copy(t.at[isrc.at[j + 3]],
                                         rbufs[nb], gsems[nb])
          dsc[j] = pltpu.async_copy(rbufs[a], acc.at[idst.at[j]],
                                    ssems[j % 2], add=True)
        dsc[GB - 1].wait()
        return carry

      lax.fori_loop(0, KB // GB, group, 0)
      plsc.subcore_barrier()
      pltpu.sync_copy(acc.at[pl.ds(row0, RPT), :],
                      outs[ch].at[core, pl.ds(row0, RPT), :])

  return agg


def _make_counts():
  mesh = plsc.VectorSubcoreMesh(core_axis_name="c", subcore_axis_name="s")
  scratch = [
      pltpu.VMEM((GB, B), jnp.int32),
      pltpu.VMEM((B, 8), jnp.float32),
      pltpu.VMEM_SHARED((ACC_ROWS, 8), jnp.float32),
  ]

  @functools.partial(pl.kernel,
                     out_type=jax.ShapeDtypeStruct((2, NP, 8), jnp.float32),
                     mesh=mesh, scratch_types=scratch,
                     compiler_params=pltpu.CompilerParams(
                         use_tc_tiling_on_sc=False))
  def cnt(dst_all, zeros8_hbm, ones8_hbm, cnt_out, idst, ones_v, cacc):
    core = lax.axis_index("c")
    s = lax.axis_index("s")
    row0 = s * RPT
    pltpu.sync_copy(ones8_hbm, ones_v)
    pltpu.sync_copy(zeros8_hbm, cacc.at[pl.ds(row0, RPT), :])
    plsc.subcore_barrier()

    def group(g, carry):
      pltpu.sync_copy(dst_all.at[core, s, pl.ds(g * GB, GB)], idst)
      for j in range(GB):
        pltpu.sync_copy(ones_v, cacc.at[idst.at[j]], add=True)
      return carry

    lax.fori_loop(0, KB // GB, group, 0)
    plsc.subcore_barrier()
    pltpu.sync_copy(cacc.at[pl.ds(row0, RPT), :],
                    cnt_out.at[core, pl.ds(row0, RPT), :])

  return cnt


def _proj(x, Wp, bp2):
  def body(x_ref, w_ref, b_ref, o0, o1, o2, o3):
    y = jnp.dot(x_ref[...], w_ref[...],
                preferred_element_type=jnp.float32) + b_ref[...]
    for c, o in enumerate((o0, o1, o2, o3)):
      o[...] = y[:, C * c:C * (c + 1)]
  return pl.pallas_call(
      body,
      grid=(N // BN,),
      in_specs=[pl.BlockSpec((BN, D), lambda i: (i, 0)),
                pl.BlockSpec((D, D), lambda i: (0, 0)),
                pl.BlockSpec((1, D), lambda i: (0, 0))],
      out_specs=[pl.BlockSpec((BN, C), lambda i: (i, 0))] * NCH,
      out_shape=[jax.ShapeDtypeStruct((N, C), jnp.float32)] * NCH,
  )(x, Wp, bp2)


def _mix1(hs, As, cnt8, W1, b1):
  def body(h0, h1, h2, h3, a0, a1, a2, a3, cnt_ref, w_ref, b_ref,
           z0, z1, z2, z3):
    cnt = cnt_ref[...]
    rp = 1.0 / jnp.maximum(cnt[0, :, 0:1], 1.0)
    rn = 1.0 / jnp.maximum(cnt[1, :, 0:1], 1.0)
    a = [r[...] for r in (a0, a1, a2, a3)]
    lhs = jnp.concatenate(
        [a[0][0] * rp, a[1][0] * rp, a[2][0] * rp, a[3][0] * rp,
         a[0][1] * rn, a[1][1] * rn, a[2][1] * rn, a[3][1] * rn,
         h0[...], h1[...], h2[...], h3[...]], axis=1)
    z = jnp.tanh(jnp.dot(lhs, w_ref[...],
                         preferred_element_type=jnp.float32) + b_ref[...])
    z0[...] = z[:, 0:C]
    z1[...] = z[:, C:2 * C]
    z2[...] = z[:, 2 * C:3 * C]
    z3[...] = z[:, 3 * C:]

  blk_h = pl.BlockSpec((BN, C), lambda i: (i, 0))
  blk_a = pl.BlockSpec((2, BN, C), lambda i: (0, i, 0))
  return pl.pallas_call(
      body,
      grid=(N // BN,),
      in_specs=[blk_h] * NCH + [blk_a] * NCH
      + [pl.BlockSpec((2, BN, 8), lambda i: (0, i, 0)),
         pl.BlockSpec((3 * D, D), lambda i: (0, 0)),
         pl.BlockSpec((1, D), lambda i: (0, 0))],
      out_specs=[blk_h] * NCH,
      out_shape=[jax.ShapeDtypeStruct((N, C), jnp.float32)] * NCH,
  )(*hs, *As, cnt8, W1, b1)


def _mix2(zs, Bs, cnt8, W2, b2):
  def body(z0, z1, z2, z3, g0, g1, g2, g3, cnt_ref, w_ref, b_ref, out_ref):
    cnt = cnt_ref[...]
    rp = 1.0 / jnp.maximum(cnt[0, :, 0:1], 1.0)
    rn = 1.0 / jnp.maximum(cnt[1, :, 0:1], 1.0)
    g = [r[...] for r in (g0, g1, g2, g3)]
    lhs = jnp.concatenate(
        [g[0][0] * rp, g[1][0] * rp, g[2][0] * rp, g[3][0] * rp,
         g[0][1] * rn, g[1][1] * rn, g[2][1] * rn, g[3][1] * rn,
         z0[...], z1[...], z2[...], z3[...]], axis=1)
    out_ref[...] = jnp.tanh(
        jnp.dot(lhs, w_ref[...], preferred_element_type=jnp.float32)
        + b_ref[...])

  blk_z = pl.BlockSpec((BN, C), lambda i: (i, 0))
  blk_a = pl.BlockSpec((2, BN, C), lambda i: (0, i, 0))
  return pl.pallas_call(
      body,
      grid=(N // BN,),
      in_specs=[blk_z] * NCH + [blk_a] * NCH
      + [pl.BlockSpec((2, BN, 8), lambda i: (0, i, 0)),
         pl.BlockSpec((3 * D, D), lambda i: (0, 0)),
         pl.BlockSpec((1, D), lambda i: (0, 0))],
      out_specs=pl.BlockSpec((BN, D), lambda i: (i, 0)),
      out_shape=jax.ShapeDtypeStruct((N, D), jnp.float32),
  )(*zs, *Bs, cnt8, W2, b2)


def kernel(x, pos_edge_index, neg_edge_index, Wp, bp,
           w1_pl, w1_pr, b1_pr, w1_nl, w1_nr, b1_nr,
           w2_pl, w2_pr, b2_pr, w2_nl, w2_nr, b2_nr):
  i32 = jnp.int32
  f32 = jnp.float32
  pad_src = jnp.zeros((PAD,), i32)
  pad_dst = jnp.full((PAD,), N, i32)
  src_all = jnp.stack([
      jnp.concatenate([pos_edge_index[0].astype(i32), pad_src]),
      jnp.concatenate([neg_edge_index[0].astype(i32), pad_src]),
  ]).reshape(2, NT, KB, B)
  dst_all = jnp.stack([
      jnp.concatenate([pos_edge_index[1].astype(i32), pad_dst]),
      jnp.concatenate([neg_edge_index[1].astype(i32), pad_dst]),
  ]).reshape(2, NT, KB, B)
  zeros32 = jnp.zeros((RPT, C), f32)
  zeros8 = jnp.zeros((RPT, 8), f32)
  ones8 = jnp.ones((B, 8), f32)

  zf = jnp.zeros((D, F2), f32)
  zh = jnp.zeros((F2, F2), f32)
  W1 = jnp.concatenate([
      jnp.concatenate([w1_pl, zf], axis=1),
      jnp.concatenate([zf, w1_nl], axis=1),
      jnp.concatenate([w1_pr, w1_nr], axis=1),
  ], axis=0)
  b1 = jnp.concatenate([b1_pr, b1_nr]).reshape(1, D)
  W2 = jnp.concatenate([
      jnp.concatenate([w2_pl[:F2], zh], axis=1),
      jnp.concatenate([zh, w2_nl[:F2]], axis=1),
      jnp.concatenate([zh, w2_nl[F2:]], axis=1),
      jnp.concatenate([w2_pl[F2:], zh], axis=1),
      jnp.concatenate([w2_pr, zh], axis=1),
      jnp.concatenate([zh, w2_nr], axis=1),
  ], axis=0)
  b2 = jnp.concatenate([b2_pr, b2_nr]).reshape(1, D)

  hs = _proj(x, Wp, bp.reshape(1, D))
  agg = _make_agg()
  cnt8 = _make_counts()(dst_all, zeros8, ones8)
  As = agg(*hs, src_all, dst_all, zeros32)
  zs = _mix1(hs, As, cnt8, W1, b1)
  Bs = agg(*zs, src_all, dst_all, zeros32)
  return _mix2(zs, Bs, cnt8, W2, b2)


# speedup vs baseline: 5.7024x; 1.0563x over previous
import functools

import jax
import jax.numpy as jnp
from jax import lax
from jax.experimental import pallas as pl
from jax.experimental.pallas import tpu as pltpu
from jax.experimental.pallas import tpu_sc as plsc

N = 50000
D = 128
F2 = 64
C = 32
NCH = 4
E = 400000
B = 128
KB = 196
NT = 16
EP = NT * KB * B
PAD = EP - E
ACC_ROWS = 50048
ZROWS = 1564
RPT = ACC_ROWS // NT
NP = ACC_ROWS
BN = 2000


GB = 14


def _make_agg():
  mesh = plsc.VectorSubcoreMesh(core_axis_name="c", subcore_axis_name="s")
  out_type = [jax.ShapeDtypeStruct((2, NP, C), jnp.float32)
              for _ in range(NCH)]
  scratch = [
      pltpu.VMEM((GB, B), jnp.int32),
      pltpu.VMEM((GB, B), jnp.int32),
      pltpu.VMEM((GB, B), jnp.int32),
      pltpu.VMEM((GB, B), jnp.int32),
  ] + [pltpu.VMEM((B, C), jnp.float32) for _ in range(5)] + [
      pltpu.VMEM_SHARED((ACC_ROWS, C), jnp.float32),
  ] + [pltpu.SemaphoreType.DMA] * 9

  @functools.partial(pl.kernel, out_type=out_type, mesh=mesh,
                     scratch_types=scratch,
                     compiler_params=pltpu.CompilerParams(
                         use_tc_tiling_on_sc=False))
  def agg(t0, t1, t2, t3, src_all, dst_all, zeros_hbm,
          o0, o1, o2, o3, isA, idA, isB, idB,
          rb0, rb1, rb2, rb3, rb4,
          acc, gs0, gs1, gs2, gs3, gs4, ss0, ss1, ss2, isem):
    outs = (o0, o1, o2, o3)
    tables = (t0, t1, t2, t3)
    rbufs = (rb0, rb1, rb2, rb3, rb4)
    gsems = (gs0, gs1, gs2, gs3, gs4)
    ssems = (ss0, ss1, ss2)
    idxbufs = ((isA, idA), (isB, idB))
    core = lax.axis_index("c")
    s = lax.axis_index("s")
    row0 = s * RPT
    ngroups = KB // GB

    def run_group(t, isrc, idst):
      dg = [None] * GB
      dsc = [None] * GB
      for p in range(3):
        dg[p] = pltpu.async_copy(t.at[isrc.at[p]], rbufs[p], gsems[p])
      for j in range(GB):
        a = j % 5
        dg[j].wait()
        if j >= 2:
          dsc[j - 2].wait()
        if j + 3 < GB:
          nb = (j + 3) % 5
          dg[j + 3] = pltpu.async_copy(t.at[isrc.at[j + 3]],
                                       rbufs[nb], gsems[nb])
        dsc[j] = pltpu.async_copy(rbufs[a], acc.at[idst.at[j]],
                                  ssems[j % 3], add=True)
      dsc[GB - 2].wait()
      dsc[GB - 1].wait()

    def fetch_idx(g, bufs):
      pltpu.async_copy(src_all.at[core, s, pl.ds(g * GB, GB)], bufs[0], isem)
      pltpu.async_copy(dst_all.at[core, s, pl.ds(g * GB, GB)], bufs[1], isem)

    def drain_idx(bufs):
      pltpu.make_async_copy(src_all.at[core, s, pl.ds(0, GB)],
                            bufs[0], isem).wait()
      pltpu.make_async_copy(src_all.at[core, s, pl.ds(0, GB)],
                            bufs[1], isem).wait()

    for ch in range(NCH):
      pltpu.sync_copy(zeros_hbm, acc.at[pl.ds(row0, RPT), :])
      if ch == 0:
        fetch_idx(0, idxbufs[0])
      plsc.subcore_barrier()
      t = tables[ch]

      def pair(p2, carry, t=t):
        for half in range(2):
          g = p2 * 2 + half
          cur = idxbufs[half]
          nxt = idxbufs[1 - half]
          drain_idx(cur)

          @pl.when(g + 1 < NCH * ngroups)
          def _():
            fetch_idx((g + 1) % ngroups, nxt)
          run_group(t, cur[0], cur[1])
        return carry

      base = ch * ngroups
      lax.fori_loop(base // 2, (base + ngroups) // 2, pair, 0)
      plsc.subcore_barrier()
      pltpu.sync_copy(acc.at[pl.ds(row0, RPT), :],
                      outs[ch].at[core, pl.ds(row0, RPT), :])

  return agg


def _make_counts():
  mesh = plsc.VectorSubcoreMesh(core_axis_name="c", subcore_axis_name="s")
  scratch = [
      pltpu.VMEM((GB, B), jnp.int32),
      pltpu.VMEM((B, 8), jnp.float32),
      pltpu.VMEM_SHARED((ACC_ROWS, 8), jnp.float32),
  ]

  @functools.partial(pl.kernel,
                     out_type=jax.ShapeDtypeStruct((2, NP, 8), jnp.float32),
                     mesh=mesh, scratch_types=scratch,
                     compiler_params=pltpu.CompilerParams(
                         use_tc_tiling_on_sc=False))
  def cnt(dst_all, zeros8_hbm, ones8_hbm, cnt_out, idst, ones_v, cacc):
    core = lax.axis_index("c")
    s = lax.axis_index("s")
    row0 = s * RPT
    pltpu.sync_copy(ones8_hbm, ones_v)
    pltpu.sync_copy(zeros8_hbm, cacc.at[pl.ds(row0, RPT), :])
    plsc.subcore_barrier()

    def group(g, carry):
      pltpu.sync_copy(dst_all.at[core, s, pl.ds(g * GB, GB)], idst)
      for j in range(GB):
        pltpu.sync_copy(ones_v, cacc.at[idst.at[j]], add=True)
      return carry

    lax.fori_loop(0, KB // GB, group, 0)
    plsc.subcore_barrier()
    pltpu.sync_copy(cacc.at[pl.ds(row0, RPT), :],
                    cnt_out.at[core, pl.ds(row0, RPT), :])

  return cnt


def _proj(x, Wp, bp2):
  def body(x_ref, w_ref, b_ref, o0, o1, o2, o3):
    y = jnp.dot(x_ref[...], w_ref[...],
                preferred_element_type=jnp.float32) + b_ref[...]
    for c, o in enumerate((o0, o1, o2, o3)):
      o[...] = y[:, C * c:C * (c + 1)]
  return pl.pallas_call(
      body,
      grid=(N // BN,),
      in_specs=[pl.BlockSpec((BN, D), lambda i: (i, 0)),
                pl.BlockSpec((D, D), lambda i: (0, 0)),
                pl.BlockSpec((1, D), lambda i: (0, 0))],
      out_specs=[pl.BlockSpec((BN, C), lambda i: (i, 0))] * NCH,
      out_shape=[jax.ShapeDtypeStruct((N, C), jnp.float32)] * NCH,
  )(x, Wp, bp2)


def _mix1(hs, As, cnt8, W1, b1):
  def body(h0, h1, h2, h3, a0, a1, a2, a3, cnt_ref, w_ref, b_ref,
           z0, z1, z2, z3):
    cnt = cnt_ref[...]
    rp = 1.0 / jnp.maximum(cnt[0, :, 0:1], 1.0)
    rn = 1.0 / jnp.maximum(cnt[1, :, 0:1], 1.0)
    a = [r[...] for r in (a0, a1, a2, a3)]
    lhs = jnp.concatenate(
        [a[0][0] * rp, a[1][0] * rp, a[2][0] * rp, a[3][0] * rp,
         a[0][1] * rn, a[1][1] * rn, a[2][1] * rn, a[3][1] * rn,
         h0[...], h1[...], h2[...], h3[...]], axis=1)
    z = jnp.tanh(jnp.dot(lhs, w_ref[...],
                         preferred_element_type=jnp.float32) + b_ref[...])
    z0[...] = z[:, 0:C]
    z1[...] = z[:, C:2 * C]
    z2[...] = z[:, 2 * C:3 * C]
    z3[...] = z[:, 3 * C:]

  blk_h = pl.BlockSpec((BN, C), lambda i: (i, 0))
  blk_a = pl.BlockSpec((2, BN, C), lambda i: (0, i, 0))
  return pl.pallas_call(
      body,
      grid=(N // BN,),
      in_specs=[blk_h] * NCH + [blk_a] * NCH
      + [pl.BlockSpec((2, BN, 8), lambda i: (0, i, 0)),
         pl.BlockSpec((3 * D, D), lambda i: (0, 0)),
         pl.BlockSpec((1, D), lambda i: (0, 0))],
      out_specs=[blk_h] * NCH,
      out_shape=[jax.ShapeDtypeStruct((N, C), jnp.float32)] * NCH,
  )(*hs, *As, cnt8, W1, b1)


def _mix2(zs, Bs, cnt8, W2, b2):
  def body(z0, z1, z2, z3, g0, g1, g2, g3, cnt_ref, w_ref, b_ref, out_ref):
    cnt = cnt_ref[...]
    rp = 1.0 / jnp.maximum(cnt[0, :, 0:1], 1.0)
    rn = 1.0 / jnp.maximum(cnt[1, :, 0:1], 1.0)
    g = [r[...] for r in (g0, g1, g2, g3)]
    lhs = jnp.concatenate(
        [g[0][0] * rp, g[1][0] * rp, g[2][0] * rp, g[3][0] * rp,
         g[0][1] * rn, g[1][1] * rn, g[2][1] * rn, g[3][1] * rn,
         z0[...], z1[...], z2[...], z3[...]], axis=1)
    out_ref[...] = jnp.tanh(
        jnp.dot(lhs, w_ref[...], preferred_element_type=jnp.float32)
        + b_ref[...])

  blk_z = pl.BlockSpec((BN, C), lambda i: (i, 0))
  blk_a = pl.BlockSpec((2, BN, C), lambda i: (0, i, 0))
  return pl.pallas_call(
      body,
      grid=(N // BN,),
      in_specs=[blk_z] * NCH + [blk_a] * NCH
      + [pl.BlockSpec((2, BN, 8), lambda i: (0, i, 0)),
         pl.BlockSpec((3 * D, D), lambda i: (0, 0)),
         pl.BlockSpec((1, D), lambda i: (0, 0))],
      out_specs=pl.BlockSpec((BN, D), lambda i: (i, 0)),
      out_shape=jax.ShapeDtypeStruct((N, D), jnp.float32),
  )(*zs, *Bs, cnt8, W2, b2)


def kernel(x, pos_edge_index, neg_edge_index, Wp, bp,
           w1_pl, w1_pr, b1_pr, w1_nl, w1_nr, b1_nr,
           w2_pl, w2_pr, b2_pr, w2_nl, w2_nr, b2_nr):
  i32 = jnp.int32
  f32 = jnp.float32
  pad_src = jnp.zeros((PAD,), i32)
  pad_dst = jnp.full((PAD,), N, i32)
  src_all = jnp.stack([
      jnp.concatenate([pos_edge_index[0].astype(i32), pad_src]),
      jnp.concatenate([neg_edge_index[0].astype(i32), pad_src]),
  ]).reshape(2, NT, KB, B)
  dst_all = jnp.stack([
      jnp.concatenate([pos_edge_index[1].astype(i32), pad_dst]),
      jnp.concatenate([neg_edge_index[1].astype(i32), pad_dst]),
  ]).reshape(2, NT, KB, B)
  zeros32 = jnp.zeros((RPT, C), f32)
  zeros8 = jnp.zeros((RPT, 8), f32)
  ones8 = jnp.ones((B, 8), f32)

  zf = jnp.zeros((D, F2), f32)
  zh = jnp.zeros((F2, F2), f32)
  W1 = jnp.concatenate([
      jnp.concatenate([w1_pl, zf], axis=1),
      jnp.concatenate([zf, w1_nl], axis=1),
      jnp.concatenate([w1_pr, w1_nr], axis=1),
  ], axis=0)
  b1 = jnp.concatenate([b1_pr, b1_nr]).reshape(1, D)
  W2 = jnp.concatenate([
      jnp.concatenate([w2_pl[:F2], zh], axis=1),
      jnp.concatenate([zh, w2_nl[:F2]], axis=1),
      jnp.concatenate([zh, w2_nl[F2:]], axis=1),
      jnp.concatenate([w2_pl[F2:], zh], axis=1),
      jnp.concatenate([w2_pr, zh], axis=1),
      jnp.concatenate([zh, w2_nr], axis=1),
  ], axis=0)
  b2 = jnp.concatenate([b2_pr, b2_nr]).reshape(1, D)

  hs = _proj(x, Wp, bp.reshape(1, D))
  agg = _make_agg()
  cnt8 = _make_counts()(dst_all, zeros8, ones8)
  As = agg(*hs, src_all, dst_all, zeros32)
  zs = _mix1(hs, As, cnt8, W1, b1)
  Bs = agg(*zs, src_all, dst_all, zeros32)
  return _mix2(zs, Bs, cnt8, W2, b2)


# speedup vs baseline: 7.1679x; 1.2570x over previous
import functools

import jax
import jax.numpy as jnp
from jax import lax
from jax.experimental import pallas as pl
from jax.experimental.pallas import tpu as pltpu
from jax.experimental.pallas import tpu_sc as plsc

N = 50000
D = 128
F2 = 64
C = 32
NCH = 4
E = 400000
B = 128
KB = 196
NT = 16
EP = NT * KB * B
PAD = EP - E
ACC_ROWS = 50048
ZROWS = 1564
RPT = ACC_ROWS // NT
NP = ACC_ROWS
BN = 2000


GB = 14


def _make_agg():
  mesh = plsc.VectorSubcoreMesh(core_axis_name="c", subcore_axis_name="s")
  out_type = jax.ShapeDtypeStruct((2, NP, D), jnp.float32)
  scratch = [
      pltpu.VMEM((GB, B), jnp.int32),
      pltpu.VMEM((GB, B), jnp.int32),
      pltpu.VMEM((GB, B), jnp.int32),
      pltpu.VMEM((GB, B), jnp.int32),
  ] + [pltpu.VMEM((B, C), jnp.float32) for _ in range(5)] + [
      pltpu.VMEM_SHARED((ACC_ROWS, C), jnp.float32),
  ] + [pltpu.SemaphoreType.DMA] * 9

  @functools.partial(pl.kernel, out_type=out_type, mesh=mesh,
                     scratch_types=scratch,
                     compiler_params=pltpu.CompilerParams(
                         use_tc_tiling_on_sc=False))
  def agg(t0, t1, t2, t3, src_all, dst_all, zeros_hbm,
          out, isA, idA, isB, idB,
          rb0, rb1, rb2, rb3, rb4,
          acc, gs0, gs1, gs2, gs3, gs4, ss0, ss1, ss2, isem):
    tables = (t0, t1, t2, t3)
    rbufs = (rb0, rb1, rb2, rb3, rb4)
    gsems = (gs0, gs1, gs2, gs3, gs4)
    ssems = (ss0, ss1, ss2)
    idxbufs = ((isA, idA), (isB, idB))
    core = lax.axis_index("c")
    s = lax.axis_index("s")
    row0 = s * RPT
    ngroups = KB // GB

    def run_group(t, isrc, idst):
      dg = [None] * GB
      dsc = [None] * GB
      for p in range(3):
        dg[p] = pltpu.async_copy(t.at[isrc.at[p]], rbufs[p], gsems[p])
      for j in range(GB):
        a = j % 5
        dg[j].wait()
        if j >= 2:
          dsc[j - 2].wait()
        if j + 3 < GB:
          nb = (j + 3) % 5
          dg[j + 3] = pltpu.async_copy(t.at[isrc.at[j + 3]],
                                       rbufs[nb], gsems[nb])
        dsc[j] = pltpu.async_copy(rbufs[a], acc.at[idst.at[j]],
                                  ssems[j % 3], add=True)
      dsc[GB - 2].wait()
      dsc[GB - 1].wait()

    def fetch_idx(g, bufs):
      pltpu.async_copy(src_all.at[core, s, pl.ds(g * GB, GB)], bufs[0], isem)
      pltpu.async_copy(dst_all.at[core, s, pl.ds(g * GB, GB)], bufs[1], isem)

    def drain_idx(bufs):
      pltpu.make_async_copy(src_all.at[core, s, pl.ds(0, GB)],
                            bufs[0], isem).wait()
      pltpu.make_async_copy(src_all.at[core, s, pl.ds(0, GB)],
                            bufs[1], isem).wait()

    for ch in range(NCH):
      pltpu.sync_copy(zeros_hbm, acc.at[pl.ds(row0, RPT), :])
      if ch == 0:
        fetch_idx(0, idxbufs[0])
      plsc.subcore_barrier()
      t = tables[ch]

      def pair(p2, carry, t=t):
        for half in range(2):
          g = p2 * 2 + half
          cur = idxbufs[half]
          nxt = idxbufs[1 - half]
          drain_idx(cur)

          @pl.when(g + 1 < NCH * ngroups)
          def _():
            fetch_idx((g + 1) % ngroups, nxt)
          run_group(t, cur[0], cur[1])
        return carry

      base = ch * ngroups
      lax.fori_loop(base // 2, (base + ngroups) // 2, pair, 0)
      plsc.subcore_barrier()
      pltpu.sync_copy(acc.at[pl.ds(row0, RPT), :],
                      out.at[core, pl.ds(row0, RPT), pl.ds(C * ch, C)])

  return agg


def _make_counts():
  mesh = plsc.VectorSubcoreMesh(core_axis_name="c", subcore_axis_name="s")
  scratch = [
      pltpu.VMEM((GB, B), jnp.int32),
      pltpu.VMEM((B, 8), jnp.float32),
      pltpu.VMEM_SHARED((ACC_ROWS, 8), jnp.float32),
  ]

  @functools.partial(pl.kernel,
                     out_type=jax.ShapeDtypeStruct((2, NP, 8), jnp.float32),
                     mesh=mesh, scratch_types=scratch,
                     compiler_params=pltpu.CompilerParams(
                         use_tc_tiling_on_sc=False))
  def cnt(dst_all, zeros8_hbm, ones8_hbm, cnt_out, idst, ones_v, cacc):
    core = lax.axis_index("c")
    s = lax.axis_index("s")
    row0 = s * RPT
    pltpu.sync_copy(ones8_hbm, ones_v)
    pltpu.sync_copy(zeros8_hbm, cacc.at[pl.ds(row0, RPT), :])
    plsc.subcore_barrier()

    def group(g, carry):
      pltpu.sync_copy(dst_all.at[core, s, pl.ds(g * GB, GB)], idst)
      for j in range(GB):
        pltpu.sync_copy(ones_v, cacc.at[idst.at[j]], add=True)
      return carry

    lax.fori_loop(0, KB // GB, group, 0)
    plsc.subcore_barrier()
    pltpu.sync_copy(cacc.at[pl.ds(row0, RPT), :],
                    cnt_out.at[core, pl.ds(row0, RPT), :])

  return cnt


def _proj(x, Wp, bp2):
  def body(x_ref, w_ref, b_ref, o0, o1, o2, o3):
    y = jnp.dot(x_ref[...], w_ref[...],
                preferred_element_type=jnp.float32) + b_ref[...]
    for c, o in enumerate((o0, o1, o2, o3)):
      o[...] = y[:, C * c:C * (c + 1)]
  return pl.pallas_call(
      body,
      grid=(N // BN,),
      in_specs=[pl.BlockSpec((BN, D), lambda i: (i, 0)),
                pl.BlockSpec((D, D), lambda i: (0, 0)),
                pl.BlockSpec((1, D), lambda i: (0, 0))],
      out_specs=[pl.BlockSpec((BN, C), lambda i: (i, 0))] * NCH,
      out_shape=[jax.ShapeDtypeStruct((N, C), jnp.float32)] * NCH,
  )(x, Wp, bp2)


def _mix1(hs, As, cnt8, W1, b1):
  def body(h0, h1, h2, h3, a_ref, cnt_ref, w_ref, b_ref,
           z0, z1, z2, z3):
    cnt = cnt_ref[...]
    rp = 1.0 / jnp.maximum(cnt[0, :, 0:1], 1.0)
    rn = 1.0 / jnp.maximum(cnt[1, :, 0:1], 1.0)
    a = a_ref[...]
    lhs = jnp.concatenate(
        [a[0] * rp, a[1] * rn,
         h0[...], h1[...], h2[...], h3[...]], axis=1)
    z = jnp.tanh(jnp.dot(lhs, w_ref[...],
                         preferred_element_type=jnp.float32) + b_ref[...])
    z0[...] = z[:, 0:C]
    z1[...] = z[:, C:2 * C]
    z2[...] = z[:, 2 * C:3 * C]
    z3[...] = z[:, 3 * C:]

  blk_h = pl.BlockSpec((BN, C), lambda i: (i, 0))
  blk_a = pl.BlockSpec((2, BN, D), lambda i: (0, i, 0))
  return pl.pallas_call(
      body,
      grid=(N // BN,),
      in_specs=[blk_h] * NCH + [blk_a]
      + [pl.BlockSpec((2, BN, 8), lambda i: (0, i, 0)),
         pl.BlockSpec((3 * D, D), lambda i: (0, 0)),
         pl.BlockSpec((1, D), lambda i: (0, 0))],
      out_specs=[blk_h] * NCH,
      out_shape=[jax.ShapeDtypeStruct((N, C), jnp.float32)] * NCH,
  )(*hs, As, cnt8, W1, b1)


def _mix2(zs, Bs, cnt8, W2, b2):
  def body(z0, z1, z2, z3, g_ref, cnt_ref, w_ref, b_ref, out_ref):
    cnt = cnt_ref[...]
    rp = 1.0 / jnp.maximum(cnt[0, :, 0:1], 1.0)
    rn = 1.0 / jnp.maximum(cnt[1, :, 0:1], 1.0)
    g = g_ref[...]
    lhs = jnp.concatenate(
        [g[0] * rp, g[1] * rn,
         z0[...], z1[...], z2[...], z3[...]], axis=1)
    out_ref[...] = jnp.tanh(
        jnp.dot(lhs, w_ref[...], preferred_element_type=jnp.float32)
        + b_ref[...])

  blk_z = pl.BlockSpec((BN, C), lambda i: (i, 0))
  blk_a = pl.BlockSpec((2, BN, D), lambda i: (0, i, 0))
  return pl.pallas_call(
      body,
      grid=(N // BN,),
      in_specs=[blk_z] * NCH + [blk_a]
      + [pl.BlockSpec((2, BN, 8), lambda i: (0, i, 0)),
         pl.BlockSpec((3 * D, D), lambda i: (0, 0)),
         pl.BlockSpec((1, D), lambda i: (0, 0))],
      out_specs=pl.BlockSpec((BN, D), lambda i: (i, 0)),
      out_shape=jax.ShapeDtypeStruct((N, D), jnp.float32),
  )(*zs, Bs, cnt8, W2, b2)


def kernel(x, pos_edge_index, neg_edge_index, Wp, bp,
           w1_pl, w1_pr, b1_pr, w1_nl, w1_nr, b1_nr,
           w2_pl, w2_pr, b2_pr, w2_nl, w2_nr, b2_nr):
  i32 = jnp.int32
  f32 = jnp.float32
  pad_src = jnp.zeros((PAD,), i32)
  pad_dst = jnp.full((PAD,), N, i32)
  src_all = jnp.stack([
      jnp.concatenate([pos_edge_index[0].astype(i32), pad_src]),
      jnp.concatenate([neg_edge_index[0].astype(i32), pad_src]),
  ]).reshape(2, NT, KB, B)
  dst_all = jnp.stack([
      jnp.concatenate([pos_edge_index[1].astype(i32), pad_dst]),
      jnp.concatenate([neg_edge_index[1].astype(i32), pad_dst]),
  ]).reshape(2, NT, KB, B)
  zeros32 = jnp.zeros((RPT, C), f32)
  zeros8 = jnp.zeros((RPT, 8), f32)
  ones8 = jnp.ones((B, 8), f32)

  zf = jnp.zeros((D, F2), f32)
  zh = jnp.zeros((F2, F2), f32)
  W1 = jnp.concatenate([
      jnp.concatenate([w1_pl, zf], axis=1),
      jnp.concatenate([zf, w1_nl], axis=1),
      jnp.concatenate([w1_pr, w1_nr], axis=1),
  ], axis=0)
  b1 = jnp.concatenate([b1_pr, b1_nr]).reshape(1, D)
  W2 = jnp.concatenate([
      jnp.concatenate([w2_pl[:F2], zh], axis=1),
      jnp.concatenate([zh, w2_nl[:F2]], axis=1),
      jnp.concatenate([zh, w2_nl[F2:]], axis=1),
      jnp.concatenate([w2_pl[F2:], zh], axis=1),
      jnp.concatenate([w2_pr, zh], axis=1),
      jnp.concatenate([zh, w2_nr], axis=1),
  ], axis=0)
  b2 = jnp.concatenate([b2_pr, b2_nr]).reshape(1, D)

  hs = _proj(x, Wp, bp.reshape(1, D))
  agg = _make_agg()
  cnt8 = _make_counts()(dst_all, zeros8, ones8)
  As = agg(*hs, src_all, dst_all, zeros32)
  zs = _mix1(hs, As, cnt8, W1, b1)
  Bs = agg(*zs, src_all, dst_all, zeros32)
  return _mix2(zs, Bs, cnt8, W2, b2)


# speedup vs baseline: 7.5004x; 1.0464x over previous
import functools

import jax
import jax.numpy as jnp
from jax import lax
from jax.experimental import pallas as pl
from jax.experimental.pallas import tpu as pltpu
from jax.experimental.pallas import tpu_sc as plsc

N = 50000
D = 128
F2 = 64
C = 32
NCH = 4
E = 400000
B = 128
KB = 196
NT = 16
EP = NT * KB * B
PAD = EP - E
ACC_ROWS = 50048
ZROWS = 1564
RPT = ACC_ROWS // NT
NP = ACC_ROWS
BN = 2000


GB = 14


def _make_agg():
  mesh = plsc.VectorSubcoreMesh(core_axis_name="c", subcore_axis_name="s")
  out_type = jax.ShapeDtypeStruct((2, NP, D), jnp.float32)
  scratch = [
      pltpu.VMEM((GB, B), jnp.int32),
      pltpu.VMEM((GB, B), jnp.int32),
      pltpu.VMEM((GB, B), jnp.int32),
      pltpu.VMEM((GB, B), jnp.int32),
  ] + [pltpu.VMEM((B, C), jnp.float32) for _ in range(5)] + [
      pltpu.VMEM_SHARED((ACC_ROWS, C), jnp.float32),
  ] + [pltpu.SemaphoreType.DMA] * 9

  @functools.partial(pl.kernel, out_type=out_type, mesh=mesh,
                     scratch_types=scratch,
                     compiler_params=pltpu.CompilerParams(
                         use_tc_tiling_on_sc=False))
  def agg(t0, t1, t2, t3, src_all, dst_all, zeros_hbm,
          out, isA, idA, isB, idB,
          rb0, rb1, rb2, rb3, rb4,
          acc, gs0, gs1, gs2, gs3, gs4, ss0, ss1, ss2, isem):
    tables = (t0, t1, t2, t3)
    rbufs = (rb0, rb1, rb2, rb3, rb4)
    gsems = (gs0, gs1, gs2, gs3, gs4)
    ssems = (ss0, ss1, ss2)
    idxbufs = ((isA, idA), (isB, idB))
    core = lax.axis_index("c")
    s = lax.axis_index("s")
    row0 = s * RPT
    ngroups = KB // GB

    def run_group(t, isrc, idst):
      dg = [None] * GB
      dsc = [None] * GB
      for p in range(4):
        dg[p] = pltpu.async_copy(t.at[isrc.at[p]], rbufs[p], gsems[p])
      for j in range(GB):
        a = j % 5
        dg[j].wait()
        if j >= 1:
          dsc[j - 1].wait()
        if j + 4 < GB:
          nb = (j + 4) % 5
          dg[j + 4] = pltpu.async_copy(t.at[isrc.at[j + 4]],
                                       rbufs[nb], gsems[nb])
        dsc[j] = pltpu.async_copy(rbufs[a], acc.at[idst.at[j]],
                                  ssems[j % 3], add=True)
      dsc[GB - 1].wait()

    def fetch_idx(g, bufs):
      base = s * KB + g * GB
      pltpu.async_copy(src_all.at[core, pl.ds(base, GB)], bufs[0], isem)
      pltpu.async_copy(dst_all.at[core, pl.ds(base, GB)], bufs[1], isem)

    def drain_idx(bufs):
      pltpu.make_async_copy(src_all.at[core, pl.ds(0, GB)],
                            bufs[0], isem).wait()
      pltpu.make_async_copy(src_all.at[core, pl.ds(0, GB)],
                            bufs[1], isem).wait()

    for ch in range(NCH):
      pltpu.sync_copy(zeros_hbm, acc.at[pl.ds(row0, RPT), :])
      if ch == 0:
        fetch_idx(0, idxbufs[0])
      plsc.subcore_barrier()
      t = tables[ch]

      def pair(p2, carry, t=t):
        for half in range(2):
          g = p2 * 2 + half
          cur = idxbufs[half]
          nxt = idxbufs[1 - half]
          drain_idx(cur)

          @pl.when(g + 1 < NCH * ngroups)
          def _():
            fetch_idx((g + 1) % ngroups, nxt)
          run_group(t, cur[0], cur[1])
        return carry

      base = ch * ngroups
      lax.fori_loop(base // 2, (base + ngroups) // 2, pair, 0)
      plsc.subcore_barrier()
      pltpu.sync_copy(acc.at[pl.ds(row0, RPT), :],
                      out.at[core, pl.ds(row0, RPT), pl.ds(C * ch, C)])

  return agg


def _make_counts():
  mesh = plsc.VectorSubcoreMesh(core_axis_name="c", subcore_axis_name="s")
  scratch = [
      pltpu.VMEM((GB, B), jnp.int32),
      pltpu.VMEM((B, 8), jnp.float32),
      pltpu.VMEM_SHARED((ACC_ROWS, 8), jnp.float32),
  ]

  @functools.partial(pl.kernel,
                     out_type=jax.ShapeDtypeStruct((2, NP, 8), jnp.float32),
                     mesh=mesh, scratch_types=scratch,
                     compiler_params=pltpu.CompilerParams(
                         use_tc_tiling_on_sc=False))
  def cnt(dst_all, zeros8_hbm, ones8_hbm, cnt_out, idst, ones_v, cacc):
    core = lax.axis_index("c")
    s = lax.axis_index("s")
    row0 = s * RPT
    pltpu.sync_copy(ones8_hbm, ones_v)
    pltpu.sync_copy(zeros8_hbm, cacc.at[pl.ds(row0, RPT), :])
    plsc.subcore_barrier()

    def group(g, carry):
      pltpu.sync_copy(dst_all.at[core, pl.ds(s * KB + g * GB, GB)], idst)
      for j in range(GB):
        pltpu.sync_copy(ones_v, cacc.at[idst.at[j]], add=True)
      return carry

    lax.fori_loop(0, KB // GB, group, 0)
    plsc.subcore_barrier()
    pltpu.sync_copy(cacc.at[pl.ds(row0, RPT), :],
                    cnt_out.at[core, pl.ds(row0, RPT), :])

  return cnt


def _proj(x, Wp, bp2):
  def body(x_ref, w_ref, b_ref, o0, o1, o2, o3):
    y = jnp.dot(x_ref[...], w_ref[...],
                preferred_element_type=jnp.float32) + b_ref[...]
    for c, o in enumerate((o0, o1, o2, o3)):
      o[...] = y[:, C * c:C * (c + 1)]
  return pl.pallas_call(
      body,
      grid=(N // BN,),
      in_specs=[pl.BlockSpec((BN, D), lambda i: (i, 0)),
                pl.BlockSpec((D, D), lambda i: (0, 0)),
                pl.BlockSpec((1, D), lambda i: (0, 0))],
      out_specs=[pl.BlockSpec((BN, C), lambda i: (i, 0))] * NCH,
      out_shape=[jax.ShapeDtypeStruct((N, C), jnp.float32)] * NCH,
  )(x, Wp, bp2)


def _mix1(hs, As, cnt8, W1, b1):
  def body(h0, h1, h2, h3, a_ref, cnt_ref, w_ref, b_ref,
           z0, z1, z2, z3):
    cnt = cnt_ref[...]
    rp = 1.0 / jnp.maximum(cnt[0, :, 0:1], 1.0)
    rn = 1.0 / jnp.maximum(cnt[1, :, 0:1], 1.0)
    a = a_ref[...]
    lhs = jnp.concatenate(
        [a[0] * rp, a[1] * rn,
         h0[...], h1[...], h2[...], h3[...]], axis=1)
    z = jnp.tanh(jnp.dot(lhs, w_ref[...],
                         preferred_element_type=jnp.float32) + b_ref[...])
    z0[...] = z[:, 0:C]
    z1[...] = z[:, C:2 * C]
    z2[...] = z[:, 2 * C:3 * C]
    z3[...] = z[:, 3 * C:]

  blk_h = pl.BlockSpec((BN, C), lambda i: (i, 0))
  return pl.pallas_call(
      body,
      grid=(N // BN,),
      in_specs=[blk_h] * NCH
      + [pl.BlockSpec((2, BN, D), lambda i: (0, i, 0)),
         pl.BlockSpec((2, BN, 8), lambda i: (0, i, 0)),
         pl.BlockSpec((3 * D, D), lambda i: (0, 0)),
         pl.BlockSpec((1, D), lambda i: (0, 0))],
      out_specs=[blk_h] * NCH,
      out_shape=[jax.ShapeDtypeStruct((N, C), jnp.float32)] * NCH,
  )(*hs, As, cnt8, W1, b1)


def _mix2(zs, Bs, cnt8, W2, b2):
  def body(z0, z1, z2, z3, g_ref, cnt_ref, w_ref, b_ref, out_ref):
    cnt = cnt_ref[...]
    rp = 1.0 / jnp.maximum(cnt[0, :, 0:1], 1.0)
    rn = 1.0 / jnp.maximum(cnt[1, :, 0:1], 1.0)
    g = g_ref[...]
    lhs = jnp.concatenate(
        [g[0] * rp, g[1] * rn,
         z0[...], z1[...], z2[...], z3[...]], axis=1)
    out_ref[...] = jnp.tanh(
        jnp.dot(lhs, w_ref[...], preferred_element_type=jnp.float32)
        + b_ref[...])

  blk_z = pl.BlockSpec((BN, C), lambda i: (i, 0))
  return pl.pallas_call(
      body,
      grid=(N // BN,),
      in_specs=[blk_z] * NCH
      + [pl.BlockSpec((2, BN, D), lambda i: (0, i, 0)),
         pl.BlockSpec((2, BN, 8), lambda i: (0, i, 0)),
         pl.BlockSpec((3 * D, D), lambda i: (0, 0)),
         pl.BlockSpec((1, D), lambda i: (0, 0))],
      out_specs=pl.BlockSpec((BN, D), lambda i: (i, 0)),
      out_shape=jax.ShapeDtypeStruct((N, D), jnp.float32),
  )(*zs, Bs, cnt8, W2, b2)


def kernel(x, pos_edge_index, neg_edge_index, Wp, bp,
           w1_pl, w1_pr, b1_pr, w1_nl, w1_nr, b1_nr,
           w2_pl, w2_pr, b2_pr, w2_nl, w2_nr, b2_nr):
  i32 = jnp.int32
  f32 = jnp.float32
  pad_src = jnp.zeros((PAD,), i32)
  pad_dst = jnp.full((PAD,), N, i32)
  src_all = jnp.stack([
      jnp.concatenate([pos_edge_index[0].astype(i32), pad_src]),
      jnp.concatenate([neg_edge_index[0].astype(i32), pad_src]),
  ]).reshape(2, NT * KB, B)
  dst_all = jnp.stack([
      jnp.concatenate([pos_edge_index[1].astype(i32), pad_dst]),
      jnp.concatenate([neg_edge_index[1].astype(i32), pad_dst]),
  ]).reshape(2, NT * KB, B)
  zeros32 = jnp.zeros((RPT, C), f32)
  zeros8 = jnp.zeros((RPT, 8), f32)
  ones8 = jnp.ones((B, 8), f32)

  zf = jnp.zeros((D, F2), f32)
  zh = jnp.zeros((F2, F2), f32)
  W1 = jnp.concatenate([
      jnp.concatenate([w1_pl, zf], axis=1),
      jnp.concatenate([zf, w1_nl], axis=1),
      jnp.concatenate([w1_pr, w1_nr], axis=1),
  ], axis=0)
  b1 = jnp.concatenate([b1_pr, b1_nr]).reshape(1, D)
  W2 = jnp.concatenate([
      jnp.concatenate([w2_pl[:F2], zh], axis=1),
      jnp.concatenate([zh, w2_nl[:F2]], axis=1),
      jnp.concatenate([zh, w2_nl[F2:]], axis=1),
      jnp.concatenate([w2_pl[F2:], zh], axis=1),
      jnp.concatenate([w2_pr, zh], axis=1),
      jnp.concatenate([zh, w2_nr], axis=1),
  ], axis=0)
  b2 = jnp.concatenate([b2_pr, b2_nr]).reshape(1, D)

  hs = _proj(x, Wp, bp.reshape(1, D))
  agg = _make_agg()
  cnt8 = _make_counts()(dst_all, zeros8, ones8)
  As = agg(*hs, src_all, dst_all, zeros32)
  zs = _mix1(hs, As, cnt8, W1, b1)
  Bs = agg(*zs, src_all, dst_all, zeros32)
  return _mix2(zs, Bs, cnt8, W2, b2)


# speedup vs baseline: 9.0040x; 1.2005x over previous
import functools

import jax
import jax.numpy as jnp
from jax import lax
from jax.experimental import pallas as pl
from jax.experimental.pallas import tpu as pltpu
from jax.experimental.pallas import tpu_sc as plsc

N = 50000
D = 128
F2 = 64
C = 32
NCH = 4
E = 400000
B = 128
KB = 196
NT = 16
EP = NT * KB * B
PAD = EP - E
ACC_ROWS = 50048
ZROWS = 1564
RPT = ACC_ROWS // NT
NP = ACC_ROWS
BN = 2000


GB = 14


def _make_agg():
  mesh = plsc.VectorSubcoreMesh(core_axis_name="c", subcore_axis_name="s")
  out_type = jax.ShapeDtypeStruct((2, NP, D), jnp.float32)
  scratch = [
      pltpu.VMEM((GB, B), jnp.int32),
      pltpu.VMEM((GB, B), jnp.int32),
      pltpu.VMEM((GB, B), jnp.int32),
      pltpu.VMEM((GB, B), jnp.int32),
  ] + [pltpu.VMEM((B, C), jnp.float32) for _ in range(5)] + [
      pltpu.VMEM_SHARED((ACC_ROWS, C), jnp.float32),
  ] + [pltpu.SemaphoreType.DMA] * 9

  @functools.partial(pl.kernel, out_type=out_type, mesh=mesh,
                     scratch_types=scratch,
                     compiler_params=pltpu.CompilerParams(
                         use_tc_tiling_on_sc=False))
  def agg(t4, src4_all, dst_all, zeros_hbm,
          out, isA, idA, isB, idB,
          rb0, rb1, rb2, rb3, rb4,
          acc, gs0, gs1, gs2, gs3, gs4, ss0, ss1, ss2, isem):
    rbufs = (rb0, rb1, rb2, rb3, rb4)
    gsems = (gs0, gs1, gs2, gs3, gs4)
    ssems = (ss0, ss1, ss2)
    idxbufs = ((isA, idA), (isB, idB))
    core = lax.axis_index("c")
    s = lax.axis_index("s")
    row0 = s * RPT
    ngroups = KB // GB

    def run_group(t, isrc, idst):
      dg = [None] * GB
      dsc = [None] * GB
      for p in range(4):
        dg[p] = pltpu.async_copy(t.at[isrc.at[p]], rbufs[p], gsems[p])
      for j in range(GB):
        a = j % 5
        dg[j].wait()
        if j >= 1:
          dsc[j - 1].wait()
        if j + 4 < GB:
          nb = (j + 4) % 5
          dg[j + 4] = pltpu.async_copy(t.at[isrc.at[j + 4]],
                                       rbufs[nb], gsems[nb])
        dsc[j] = pltpu.async_copy(rbufs[a], acc.at[idst.at[j]],
                                  ssems[j % 3], add=True)
      dsc[GB - 1].wait()

    def fetch_idx(g, bufs):
      base = s * KB + (g % ngroups) * GB
      pltpu.async_copy(src4_all.at[g // ngroups, core, pl.ds(base, GB)],
                       bufs[0], isem)
      pltpu.async_copy(dst_all.at[core, pl.ds(base, GB)], bufs[1], isem)

    def drain_idx(bufs):
      pltpu.make_async_copy(dst_all.at[core, pl.ds(0, GB)],
                            bufs[0], isem).wait()
      pltpu.make_async_copy(dst_all.at[core, pl.ds(0, GB)],
                            bufs[1], isem).wait()

    for ch in range(NCH):
      pltpu.sync_copy(zeros_hbm, acc.at[pl.ds(row0, RPT), :])
      if ch == 0:
        fetch_idx(0, idxbufs[0])
      plsc.subcore_barrier()

      def pair(p2, carry):
        for half in range(2):
          g = p2 * 2 + half
          cur = idxbufs[half]
          nxt = idxbufs[1 - half]
          drain_idx(cur)

          @pl.when(g + 1 < NCH * ngroups)
          def _():
            fetch_idx(g + 1, nxt)
          run_group(t4, cur[0], cur[1])
        return carry

      base = ch * ngroups
      lax.fori_loop(base // 2, (base + ngroups) // 2, pair, 0)
      plsc.subcore_barrier()
      pltpu.sync_copy(acc.at[pl.ds(row0, RPT), :],
                      out.at[core, pl.ds(row0, RPT), pl.ds(C * ch, C)])

  return agg


def _make_counts():
  mesh = plsc.VectorSubcoreMesh(core_axis_name="c", subcore_axis_name="s")
  scratch = [
      pltpu.VMEM((GB, B), jnp.int32),
      pltpu.VMEM((B, 8), jnp.float32),
      pltpu.VMEM_SHARED((ACC_ROWS, 8), jnp.float32),
  ]

  @functools.partial(pl.kernel,
                     out_type=jax.ShapeDtypeStruct((2, NP, 8), jnp.float32),
                     mesh=mesh, scratch_types=scratch,
                     compiler_params=pltpu.CompilerParams(
                         use_tc_tiling_on_sc=False))
  def cnt(dst_all, zeros8_hbm, ones8_hbm, cnt_out, idst, ones_v, cacc):
    core = lax.axis_index("c")
    s = lax.axis_index("s")
    row0 = s * RPT
    pltpu.sync_copy(ones8_hbm, ones_v)
    pltpu.sync_copy(zeros8_hbm, cacc.at[pl.ds(row0, RPT), :])
    plsc.subcore_barrier()

    def group(g, carry):
      pltpu.sync_copy(dst_all.at[core, pl.ds(s * KB + g * GB, GB)], idst)
      for j in range(GB):
        pltpu.sync_copy(ones_v, cacc.at[idst.at[j]], add=True)
      return carry

    lax.fori_loop(0, KB // GB, group, 0)
    plsc.subcore_barrier()
    pltpu.sync_copy(cacc.at[pl.ds(row0, RPT), :],
                    cnt_out.at[core, pl.ds(row0, RPT), :])

  return cnt


def _proj(x, Wp, bp2):
  def body(x_ref, w_ref, b_ref, o_ref):
    o_ref[...] = jnp.dot(x_ref[...], w_ref[...],
                         preferred_element_type=jnp.float32) + b_ref[...]
  return pl.pallas_call(
      body,
      grid=(N // BN,),
      in_specs=[pl.BlockSpec((BN, D), lambda i: (i, 0)),
                pl.BlockSpec((D, D), lambda i: (0, 0)),
                pl.BlockSpec((1, D), lambda i: (0, 0))],
      out_specs=pl.BlockSpec((BN, D), lambda i: (i, 0)),
      out_shape=jax.ShapeDtypeStruct((N, D), jnp.float32),
  )(x, Wp, bp2)


def _mix1(hs, As, cnt8, W1, b1):
  def body(h_ref, a_ref, cnt_ref, w_ref, b_ref, z_ref):
    cnt = cnt_ref[...]
    rp = 1.0 / jnp.maximum(cnt[0, :, 0:1], 1.0)
    rn = 1.0 / jnp.maximum(cnt[1, :, 0:1], 1.0)
    a = a_ref[...]
    lhs = jnp.concatenate([a[0] * rp, a[1] * rn, h_ref[...]], axis=1)
    z_ref[...] = jnp.tanh(
        jnp.dot(lhs, w_ref[...], preferred_element_type=jnp.float32)
        + b_ref[...])

  return pl.pallas_call(
      body,
      grid=(N // BN,),
      in_specs=[pl.BlockSpec((BN, D), lambda i: (i, 0)),
                pl.BlockSpec((2, BN, D), lambda i: (0, i, 0)),
                pl.BlockSpec((2, BN, 8), lambda i: (0, i, 0)),
                pl.BlockSpec((3 * D, D), lambda i: (0, 0)),
                pl.BlockSpec((1, D), lambda i: (0, 0))],
      out_specs=pl.BlockSpec((BN, D), lambda i: (i, 0)),
      out_shape=jax.ShapeDtypeStruct((N, D), jnp.float32),
  )(hs, As, cnt8, W1, b1)


def _mix2(zs, Bs, cnt8, W2, b2):
  def body(z_ref, g_ref, cnt_ref, w_ref, b_ref, out_ref):
    cnt = cnt_ref[...]
    rp = 1.0 / jnp.maximum(cnt[0, :, 0:1], 1.0)
    rn = 1.0 / jnp.maximum(cnt[1, :, 0:1], 1.0)
    g = g_ref[...]
    lhs = jnp.concatenate([g[0] * rp, g[1] * rn, z_ref[...]], axis=1)
    out_ref[...] = jnp.tanh(
        jnp.dot(lhs, w_ref[...], preferred_element_type=jnp.float32)
        + b_ref[...])

  return pl.pallas_call(
      body,
      grid=(N // BN,),
      in_specs=[pl.BlockSpec((BN, D), lambda i: (i, 0)),
                pl.BlockSpec((2, BN, D), lambda i: (0, i, 0)),
                pl.BlockSpec((2, BN, 8), lambda i: (0, i, 0)),
                pl.BlockSpec((3 * D, D), lambda i: (0, 0)),
                pl.BlockSpec((1, D), lambda i: (0, 0))],
      out_specs=pl.BlockSpec((BN, D), lambda i: (i, 0)),
      out_shape=jax.ShapeDtypeStruct((N, D), jnp.float32),
  )(zs, Bs, cnt8, W2, b2)


def kernel(x, pos_edge_index, neg_edge_index, Wp, bp,
           w1_pl, w1_pr, b1_pr, w1_nl, w1_nr, b1_nr,
           w2_pl, w2_pr, b2_pr, w2_nl, w2_nr, b2_nr):
  i32 = jnp.int32
  f32 = jnp.float32
  pad_src = jnp.zeros((PAD,), i32)
  pad_dst = jnp.full((PAD,), N, i32)
  src_all = jnp.stack([
      jnp.concatenate([pos_edge_index[0].astype(i32), pad_src]),
      jnp.concatenate([neg_edge_index[0].astype(i32), pad_src]),
  ]).reshape(2, NT * KB, B)
  dst_all = jnp.stack([
      jnp.concatenate([pos_edge_index[1].astype(i32), pad_dst]),
      jnp.concatenate([neg_edge_index[1].astype(i32), pad_dst]),
  ]).reshape(2, NT * KB, B)
  zeros32 = jnp.zeros((RPT, C), f32)
  zeros8 = jnp.zeros((RPT, 8), f32)
  ones8 = jnp.ones((B, 8), f32)

  zf = jnp.zeros((D, F2), f32)
  zh = jnp.zeros((F2, F2), f32)
  W1 = jnp.concatenate([
      jnp.concatenate([w1_pl, zf], axis=1),
      jnp.concatenate([zf, w1_nl], axis=1),
      jnp.concatenate([w1_pr, w1_nr], axis=1),
  ], axis=0)
  b1 = jnp.concatenate([b1_pr, b1_nr]).reshape(1, D)
  W2 = jnp.concatenate([
      jnp.concatenate([w2_pl[:F2], zh], axis=1),
      jnp.concatenate([zh, w2_nl[:F2]], axis=1),
      jnp.concatenate([zh, w2_nl[F2:]], axis=1),
      jnp.concatenate([w2_pl[F2:], zh], axis=1),
      jnp.concatenate([w2_pr, zh], axis=1),
      jnp.concatenate([zh, w2_nr], axis=1),
  ], axis=0)
  b2 = jnp.concatenate([b2_pr, b2_nr]).reshape(1, D)

  src4_all = 4 * src_all[None] + jnp.arange(NCH, dtype=i32)[:, None, None,
                                                            None]
  h = _proj(x, Wp, bp.reshape(1, D))
  agg = _make_agg()
  cnt8 = _make_counts()(dst_all, zeros8, ones8)
  As = agg(jnp.reshape(h, (NCH * N, C)), src4_all, dst_all, zeros32)
  z = _mix1(h, As, cnt8, W1, b1)
  Bs = agg(jnp.reshape(z, (NCH * N, C)), src4_all, dst_all, zeros32)
  return _mix2(z, Bs, cnt8, W2, b2)
